# Initial kernel scaffold; baseline (speedup 1.0000x reference)
import jax
import jax.numpy as jnp
from jax.experimental import pallas as pl


def kernel(x, w_gate, w_noise, W1, b1, W2, b2):
    raise NotImplementedError("write your pallas kernel here")


# speedup vs baseline: 3.3559x; 3.3559x over previous
import functools

import jax
import jax.numpy as jnp
from jax import lax
from jax.experimental import pallas as pl
from jax.experimental.pallas import tpu as pltpu
from jax.experimental.pallas import tpu_sc as plsc

E = 8
K = 2
D = 1024
H = 4096
N = 2048

BR = 128
PAD = 5120
NT = PAD // BR
NW = 32
TPW = N // NW
CPW = 32

_SQRT1_2 = 0.7071067811865476


def _gelu_exact(v):
    return 0.5 * v * (1.0 + lax.erf(v * _SQRT1_2))


def _gating_body(x_ref, wg_ref, pos0_ref, pos1_ref, g0_ref, g1_ref,
                 te_ref, loss_ref):
    xf = x_ref[...]
    wg = wg_ref[...]
    logits = jnp.dot(xf, wg, preferred_element_type=jnp.float32)

    iota_e = lax.broadcasted_iota(jnp.int32, (N, E), 1)
    m1 = jnp.max(logits, axis=1, keepdims=True)
    a1 = jnp.min(jnp.where(logits == m1, iota_e, E), axis=1, keepdims=True)
    oh1 = iota_e == a1
    l2 = jnp.where(oh1, -jnp.inf, logits)
    m2 = jnp.max(l2, axis=1, keepdims=True)
    a2 = jnp.min(jnp.where(l2 == m2, iota_e, E), axis=1, keepdims=True)
    oh2 = iota_e == a2

    e2 = jnp.exp(m2 - m1)
    denom = 1.0 + e2
    gate1 = 1.0 / denom
    gate2 = e2 / denom

    gates_dense = (jnp.where(oh1, gate1, 0.0)
                   + jnp.where(oh2, gate2, 0.0))

    imp = jnp.sum(gates_dense, axis=0, keepdims=True)
    loadv = jnp.sum((gates_dense > 0.0).astype(jnp.float32),
                    axis=0, keepdims=True)

    def cv2(v):
        m = jnp.mean(v)
        var = jnp.sum((v - m) ** 2) / (E - 1)
        return var / (m * m + 1e-10)

    loss_ref[0, 0] = (cv2(imp) + cv2(loadv)) * 1e-2

    occ = oh1.astype(jnp.float32) + oh2.astype(jnp.float32)
    inc = occ
    s = 1
    while s < N:
        inc = inc + jnp.concatenate(
            [jnp.zeros((s, E), jnp.float32), inc[:-s]], axis=0)
        s *= 2
    cex = inc - occ

    off = jnp.int32(0)
    offs = []
    ends = []
    for e in range(E):
        ce = jnp.sum(occ[:, e:e + 1]).astype(jnp.int32)
        cpe = ((ce + BR - 1) // BR) * BR
        offs.append(off)
        off = off + cpe
        ends.append(off)

    iota_row = lax.broadcasted_iota(jnp.int32, (1, E), 1)
    offs_vec = jnp.zeros((1, E), jnp.float32)
    for e in range(E):
        offs_vec = offs_vec + jnp.where(iota_row == e,
                                        offs[e].astype(jnp.float32), 0.0)

    slot = offs_vec + cex
    p0 = jnp.sum(jnp.where(oh1, slot, 0.0), axis=1, keepdims=True)
    p1 = jnp.sum(jnp.where(oh2, slot, 0.0), axis=1, keepdims=True)
    pos0_ref[...] = p0.astype(jnp.int32)
    pos1_ref[...] = p1.astype(jnp.int32)
    ones16 = jnp.ones((1, 16), jnp.float32)
    g0_ref[...] = gate1 * ones16
    g1_ref[...] = gate2 * ones16

    for i in range(NT):
        t = 0
        for e in range(E):
            t = t + jnp.where(ends[e] <= i * BR, 1, 0).astype(jnp.int32)
        te_ref[i] = jnp.minimum(t, E - 1)
    te_ref[NT] = off // BR
    for i in range(NT + 1, 64):
        te_ref[i] = 0


_gating = pl.pallas_call(
    _gating_body,
    out_shape=(
        jax.ShapeDtypeStruct((N, 1), jnp.int32),
        jax.ShapeDtypeStruct((N, 1), jnp.int32),
        jax.ShapeDtypeStruct((N, 16), jnp.float32),
        jax.ShapeDtypeStruct((N, 16), jnp.float32),
        jax.ShapeDtypeStruct((64,), jnp.int32),
        jax.ShapeDtypeStruct((1, 1), jnp.float32),
    ),
    out_specs=(
        pl.BlockSpec(memory_space=pltpu.VMEM),
        pl.BlockSpec(memory_space=pltpu.VMEM),
        pl.BlockSpec(memory_space=pltpu.VMEM),
        pl.BlockSpec(memory_space=pltpu.VMEM),
        pl.BlockSpec(memory_space=pltpu.SMEM),
        pl.BlockSpec(memory_space=pltpu.SMEM),
    ),
)


@functools.lru_cache(maxsize=None)
def _build_dispatch():
    mesh = plsc.VectorSubcoreMesh(core_axis_name="c", subcore_axis_name="s")

    @functools.partial(
        pl.kernel,
        mesh=mesh,
        out_type=jax.ShapeDtypeStruct((PAD, D), jnp.float32),
        scratch_types=[
            pltpu.VMEM((TPW, D), jnp.float32),
            pltpu.VMEM((TPW,), jnp.int32),
            pltpu.VMEM((TPW,), jnp.int32),
            pltpu.SemaphoreType.DMA,
            pltpu.SemaphoreType.DMA,
        ],
    )
    def dispatch_kernel(x_hbm, pos0_hbm, pos1_hbm, xs_hbm, xv, i0v, i1v,
                        s0, s1):
        wid = lax.axis_index("s") * 2 + lax.axis_index("c")
        base = wid * TPW
        pltpu.sync_copy(x_hbm.at[pl.ds(base, TPW)], xv)
        pltpu.sync_copy(pos0_hbm.at[pl.ds(base, TPW)], i0v)
        pltpu.sync_copy(pos1_hbm.at[pl.ds(base, TPW)], i1v)
        c0 = pltpu.async_copy(xv, xs_hbm.at[i0v], s0)
        c1 = pltpu.async_copy(xv, xs_hbm.at[i1v], s1)
        c0.wait()
        c1.wait()

    return dispatch_kernel


def _dispatch(x, pos0, pos1):
    return _build_dispatch()(x, pos0, pos1)


def _mm1_body(te_ref, xs_ref, w1_ref, b1_ref, h_ref):
    i = pl.program_id(0)

    @pl.when(i < te_ref[NT])
    def _():
        acc = jnp.dot(xs_ref[...], w1_ref[0],
                      preferred_element_type=jnp.float32)
        h_ref[...] = _gelu_exact(acc + b1_ref[0])


_mm1 = pl.pallas_call(
    _mm1_body,
    grid_spec=pltpu.PrefetchScalarGridSpec(
        num_scalar_prefetch=1,
        grid=(NT,),
        in_specs=[
            pl.BlockSpec((BR, D), lambda i, te: (i, 0)),
            pl.BlockSpec((1, D, H), lambda i, te: (te[i], 0, 0)),
            pl.BlockSpec((1, 1, H), lambda i, te: (te[i], 0, 0)),
        ],
        out_specs=pl.BlockSpec((BR, H), lambda i, te: (i, 0)),
    ),
    out_shape=jax.ShapeDtypeStruct((PAD, H), jnp.float32),
)


def _mm2_body(te_ref, h_ref, w2_ref, b2_ref, o_ref):
    i = pl.program_id(0)

    @pl.when(i < te_ref[NT])
    def _():
        acc = jnp.dot(h_ref[...], w2_ref[0],
                      preferred_element_type=jnp.float32)
        o_ref[...] = acc + b2_ref[0]


_mm2 = pl.pallas_call(
    _mm2_body,
    grid_spec=pltpu.PrefetchScalarGridSpec(
        num_scalar_prefetch=1,
        grid=(NT,),
        in_specs=[
            pl.BlockSpec((BR, H), lambda i, te: (i, 0)),
            pl.BlockSpec((1, H, D), lambda i, te: (te[i], 0, 0)),
            pl.BlockSpec((1, 1, D), lambda i, te: (te[i], 0, 0)),
        ],
        out_specs=pl.BlockSpec((BR, D), lambda i, te: (i, 0)),
    ),
    out_shape=jax.ShapeDtypeStruct((PAD, D), jnp.float32),
)


@functools.lru_cache(maxsize=None)
def _build_combine():
    mesh = plsc.VectorSubcoreMesh(core_axis_name="c", subcore_axis_name="s")

    @functools.partial(
        pl.kernel,
        mesh=mesh,
        out_type=jax.ShapeDtypeStruct((N, D), jnp.float32),
        scratch_types=[
            pltpu.VMEM((CPW, D), jnp.float32),
            pltpu.VMEM((CPW, D), jnp.float32),
            pltpu.VMEM((CPW, D), jnp.float32),
            pltpu.VMEM((CPW,), jnp.int32),
            pltpu.VMEM((CPW,), jnp.int32),
            pltpu.VMEM((CPW, 16), jnp.float32),
            pltpu.VMEM((CPW, 16), jnp.float32),
            pltpu.SemaphoreType.DMA,
            pltpu.SemaphoreType.DMA,
        ],
    )
    def combine_kernel(o_hbm, pos0_hbm, pos1_hbm, g0_hbm, g1_hbm, y_hbm,
                       r0v, r1v, yv, i0v, i1v, g0v, g1v, s0, s1):
        wid = lax.axis_index("s") * 2 + lax.axis_index("c")
        for ch in range(TPW // CPW):
            base = wid * TPW + ch * CPW
            pltpu.sync_copy(pos0_hbm.at[pl.ds(base, CPW)], i0v)
            pltpu.sync_copy(pos1_hbm.at[pl.ds(base, CPW)], i1v)
            pltpu.sync_copy(g0_hbm.at[pl.ds(base, CPW)], g0v)
            pltpu.sync_copy(g1_hbm.at[pl.ds(base, CPW)], g1v)
            c0 = pltpu.async_copy(o_hbm.at[i0v], r0v, s0)
            c1 = pltpu.async_copy(o_hbm.at[i1v], r1v, s1)
            c0.wait()
            c1.wait()

            def row_body(r, carry):
                ga = g0v[r]
                gb = g1v[r]

                def col_body(c, carry2):
                    sl = pl.ds(c * 16, 16)
                    yv[r, sl] = ga * r0v[r, sl] + gb * r1v[r, sl]
                    return carry2

                return lax.fori_loop(0, D // 16, col_body, carry, unroll=8)

            lax.fori_loop(0, CPW, row_body, 0)
            pltpu.sync_copy(yv, y_hbm.at[pl.ds(base, CPW)])

    return combine_kernel


def _combine(o, pos0, pos1, g0, g1):
    return _build_combine()(o, pos0, pos1, g0, g1)


def kernel(x, w_gate, w_noise, W1, b1, W2, b2):
    pos0, pos1, g0, g1, te, loss = _gating(x, w_gate)
    pos0 = pos0.reshape(N)
    pos1 = pos1.reshape(N)
    xs = _dispatch(x, pos0, pos1)
    h = _mm1(te, xs, W1, b1.reshape(E, 1, H))
    o = _mm2(te, h, W2, b2.reshape(E, 1, D))
    y = _combine(o, pos0, pos1, g0, g1)
    return y, loss[0, 0]
